# speedup vs baseline: 2.3305x; 2.3305x over previous
import functools

import jax
import jax.numpy as jnp
from jax import lax
from jax.experimental import pallas as pl
from jax.experimental.pallas import tpu as pltpu
from jax.experimental.pallas import tpu_sc as plsc

_NC = 2
_NS = 16
_NW = _NC * _NS


def _sc_gather(table, ids, d_model):
    b = ids.shape[0]
    b_per_w = b // _NW
    chunk = 64
    n_chunks = b_per_w // chunk
    mesh = plsc.VectorSubcoreMesh(
        core_axis_name="c", subcore_axis_name="s", num_cores=_NC,
        num_subcores=_NS)

    @functools.partial(
        pl.kernel,
        out_type=jax.ShapeDtypeStruct((b, d_model), jnp.float32),
        mesh=mesh,
        scratch_types=[
            pltpu.VMEM((2, chunk), jnp.int32),
            pltpu.VMEM((2, chunk, d_model), jnp.float32),
            pltpu.SemaphoreType.DMA,
            pltpu.SemaphoreType.DMA,
        ],
    )
    def k(table_hbm, idx_hbm, out_hbm, idx_v, rows_v, gsem0, gsem1):
        wid = lax.axis_index("s") * _NC + lax.axis_index("c")
        base = wid * b_per_w
        gsems = (gsem0, gsem1)

        def start(c, slot):
            pltpu.sync_copy(idx_hbm.at[pl.ds(base + c * chunk, chunk)],
                            idx_v.at[slot])
            return pltpu.async_copy(table_hbm.at[idx_v.at[slot]],
                                    rows_v.at[slot], gsems[slot])

        pending = start(0, 0)
        for c in range(n_chunks):
            slot = c % 2
            nxt = None
            if c + 1 < n_chunks:
                nxt = start(c + 1, (c + 1) % 2)
            pending.wait()
            pltpu.sync_copy(rows_v.at[slot],
                            out_hbm.at[pl.ds(base + c * chunk, chunk)])
            pending = nxt

    return k(table, ids)


def _tc_body(x_ref, g_ref, w_ref, b_ref, rg_ref, rb_ref, cg_ref, cb_ref,
             cat_ref, ratio_ref):
    eps = 1e-5

    def ln_relu(v, gamma, beta):
        mu = jnp.mean(v, axis=-1, keepdims=True)
        var = jnp.mean(jnp.square(v - mu), axis=-1, keepdims=True)
        y = (v - mu) * lax.rsqrt(var + eps) * gamma + beta
        return jnp.maximum(y, 0.0)

    r = lax.dot_general(x_ref[...], w_ref[...], (((1,), (1,)), ((), ())),
                        preferred_element_type=jnp.float32)
    r = r + b_ref[...]
    ratio_ref[...] = ln_relu(r, rg_ref[...], rb_ref[...])
    cat_ref[...] = ln_relu(g_ref[...], cg_ref[...], cb_ref[...])


def kernel(canvas_cat_ids, canvas_ratio_feat, cat_table, cat_ln_g, cat_ln_b,
           ratio_W, ratio_b, ratio_ln_g, ratio_ln_b):
    b, in_feat = canvas_ratio_feat.shape
    d_model = cat_table.shape[1]
    ids = canvas_cat_ids.astype(jnp.int32)

    gathered = _sc_gather(cat_table, ids, d_model)

    bn = 1024
    grid = (b // bn,)
    row_spec = pl.BlockSpec((bn, d_model), lambda i: (i, 0))
    vec_spec = pl.BlockSpec((1, d_model), lambda i: (0, 0))
    full_w = pl.BlockSpec((d_model, in_feat), lambda i: (0, 0))

    cat, ratio = pl.pallas_call(
        _tc_body,
        grid=grid,
        in_specs=[
            pl.BlockSpec((bn, in_feat), lambda i: (i, 0)),
            row_spec,
            full_w,
            vec_spec, vec_spec, vec_spec, vec_spec, vec_spec,
        ],
        out_specs=[row_spec, row_spec],
        out_shape=[
            jax.ShapeDtypeStruct((b, d_model), jnp.float32),
            jax.ShapeDtypeStruct((b, d_model), jnp.float32),
        ],
    )(
        canvas_ratio_feat, gathered, ratio_W,
        ratio_b.reshape(1, -1), ratio_ln_g.reshape(1, -1),
        ratio_ln_b.reshape(1, -1), cat_ln_g.reshape(1, -1),
        cat_ln_b.reshape(1, -1),
    )
    return (cat, ratio)


# speedup vs baseline: 2.3819x; 1.0220x over previous
import functools

import jax
import jax.numpy as jnp
from jax import lax
from jax.experimental import pallas as pl
from jax.experimental.pallas import tpu as pltpu
from jax.experimental.pallas import tpu_sc as plsc

_NC = 2
_NS = 16
_NW = _NC * _NS


def _sc_gather(table, ids, d_model):
    b = ids.shape[0]
    b_per_w = b // _NW
    chunk = 64
    n_chunks = b_per_w // chunk
    mesh = plsc.VectorSubcoreMesh(
        core_axis_name="c", subcore_axis_name="s", num_cores=_NC,
        num_subcores=_NS)

    @functools.partial(
        pl.kernel,
        out_type=jax.ShapeDtypeStruct((b, d_model), jnp.float32),
        mesh=mesh,
        scratch_types=[
            pltpu.VMEM((2, chunk), jnp.int32),
            pltpu.VMEM((2, chunk, d_model), jnp.float32),
            pltpu.SemaphoreType.DMA,
            pltpu.SemaphoreType.DMA,
        ],
    )
    def k(table_hbm, idx_hbm, out_hbm, idx_v, rows_v, gsem0, gsem1):
        wid = lax.axis_index("s") * _NC + lax.axis_index("c")
        base = wid * b_per_w
        gsems = (gsem0, gsem1)

        def start(c, slot):
            pltpu.sync_copy(idx_hbm.at[pl.ds(base + c * chunk, chunk)],
                            idx_v.at[slot])
            return pltpu.async_copy(table_hbm.at[idx_v.at[slot]],
                                    rows_v.at[slot], gsems[slot])

        pending = start(0, 0)
        for c in range(n_chunks):
            slot = c % 2
            nxt = None
            if c + 1 < n_chunks:
                nxt = start(c + 1, (c + 1) % 2)
            pending.wait()
            pltpu.sync_copy(rows_v.at[slot],
                            out_hbm.at[pl.ds(base + c * chunk, chunk)])
            pending = nxt

    return k(table, ids)


def _ln_relu(v, gamma, beta):
    eps = 1e-5
    mu = jnp.mean(v, axis=-1, keepdims=True)
    var = jnp.mean(jnp.square(v - mu), axis=-1, keepdims=True)
    y = (v - mu) * lax.rsqrt(var + eps) * gamma + beta
    return jnp.maximum(y, 0.0)


def _tc_ratio_body(x_ref, w_ref, b_ref, rg_ref, rb_ref, ratio_ref):
    r = lax.dot_general(x_ref[...], w_ref[...], (((1,), (1,)), ((), ())),
                        preferred_element_type=jnp.float32)
    r = r + b_ref[...]
    ratio_ref[...] = _ln_relu(r, rg_ref[...], rb_ref[...])


def _tc_cat_body(g_ref, cg_ref, cb_ref, cat_ref):
    cat_ref[...] = _ln_relu(g_ref[...], cg_ref[...], cb_ref[...])


def kernel(canvas_cat_ids, canvas_ratio_feat, cat_table, cat_ln_g, cat_ln_b,
           ratio_W, ratio_b, ratio_ln_g, ratio_ln_b):
    b, in_feat = canvas_ratio_feat.shape
    d_model = cat_table.shape[1]
    ids = canvas_cat_ids.astype(jnp.int32)

    gathered = _sc_gather(cat_table, ids, d_model)

    bn = 1024
    grid = (b // bn,)
    row_spec = pl.BlockSpec((bn, d_model), lambda i: (i, 0))
    vec_spec = pl.BlockSpec((1, d_model), lambda i: (0, 0))
    full_w = pl.BlockSpec((d_model, in_feat), lambda i: (0, 0))

    ratio = pl.pallas_call(
        _tc_ratio_body,
        grid=grid,
        in_specs=[
            pl.BlockSpec((bn, in_feat), lambda i: (i, 0)),
            full_w,
            vec_spec, vec_spec, vec_spec,
        ],
        out_specs=row_spec,
        out_shape=jax.ShapeDtypeStruct((b, d_model), jnp.float32),
    )(
        canvas_ratio_feat, ratio_W,
        ratio_b.reshape(1, -1), ratio_ln_g.reshape(1, -1),
        ratio_ln_b.reshape(1, -1),
    )
    cat = pl.pallas_call(
        _tc_cat_body,
        grid=grid,
        in_specs=[row_spec, vec_spec, vec_spec],
        out_specs=row_spec,
        out_shape=jax.ShapeDtypeStruct((b, d_model), jnp.float32),
    )(gathered, cat_ln_g.reshape(1, -1), cat_ln_b.reshape(1, -1))
    return (cat, ratio)
